# baseline (speedup 1.0000x reference)
import functools

import jax
import jax.numpy as jnp
from jax.experimental import pallas as pl
from jax.experimental.pallas import tpu as pltpu

MODEL_DIM = 128
MAX_LENGTH = 128
BATCH = 4096
SEQ_LEN = 128
TOTAL_ROWS = BATCH * SEQ_LEN
R = 512
NBLK = TOTAL_ROWS // R


def _body(idx_ref, table_ref, out_ref):
    idx = idx_ref[0, 0, :]
    oh = (jax.lax.broadcasted_iota(jnp.int32, (R, MAX_LENGTH), 1)
          == idx[:, None]).astype(jnp.float32)
    out_ref[...] = jnp.dot(oh, table_ref[...],
                           preferred_element_type=jnp.float32)


@jax.jit
def kernel(span_indices, table):
    idx = span_indices.reshape(NBLK, 1, R).astype(jnp.int32)
    out = pl.pallas_call(
        _body,
        grid=(NBLK,),
        in_specs=[
            pl.BlockSpec((1, 1, R), lambda i: (i, 0, 0)),
            pl.BlockSpec((MAX_LENGTH, MODEL_DIM), lambda i: (0, 0)),
        ],
        out_specs=pl.BlockSpec((R, MODEL_DIM), lambda i: (i, 0)),
        out_shape=jax.ShapeDtypeStruct((TOTAL_ROWS, MODEL_DIM), jnp.float32),
        compiler_params=pltpu.CompilerParams(
            dimension_semantics=("arbitrary",),
        ),
    )(idx, table)
    return out.reshape(BATCH, SEQ_LEN, MODEL_DIM)


# speedup vs baseline: 4.7606x; 4.7606x over previous
import functools

import jax
import jax.numpy as jnp
from jax import lax
from jax.experimental import pallas as pl
from jax.experimental.pallas import tpu as pltpu
from jax.experimental.pallas import tpu_sc as plsc

MODEL_DIM = 128
MAX_LENGTH = 128
BATCH = 4096
SEQ_LEN = 128

_INFO = plsc.get_sparse_core_info()
NC = _INFO.num_cores
NS = _INFO.num_subcores
NW = NC * NS
TOTAL_ROWS = BATCH * SEQ_LEN
ROWS_PER_W = TOTAL_ROWS // NW
CHUNK = 128
NCHUNKS = ROWS_PER_W // CHUNK


def _make_kernel():
    mesh = plsc.VectorSubcoreMesh(core_axis_name="c", subcore_axis_name="s")

    @functools.partial(
        pl.kernel,
        mesh=mesh,
        out_type=jax.ShapeDtypeStruct((TOTAL_ROWS, MODEL_DIM), jnp.float32),
        scratch_types=[
            pltpu.VMEM((NCHUNKS, CHUNK), jnp.int32),
            pltpu.VMEM((4, CHUNK, MODEL_DIM), jnp.float32),
            pltpu.VMEM_SHARED((MAX_LENGTH, MODEL_DIM), jnp.float32),
            pltpu.SemaphoreType.DMA,
            pltpu.SemaphoreType.DMA,
        ],
    )
    def gather_kernel(idx_hbm, table_hbm, out_hbm, idx_v, rows_v, table_sh,
                      g_sem, w_sem):
        c = lax.axis_index("c")
        s = lax.axis_index("s")
        wid = s * NC + c
        base = wid * ROWS_PER_W

        @pl.when(s == 0)
        def _():
            pltpu.sync_copy(table_hbm, table_sh)

        pltpu.sync_copy(idx_hbm.at[wid], idx_v)
        plsc.subcore_barrier()

        pltpu.async_copy(table_sh.at[idx_v.at[0]], rows_v.at[0], g_sem)
        pltpu.async_copy(table_sh.at[idx_v.at[1]], rows_v.at[1], g_sem)

        def chunk_step(i, carry):
            pltpu.make_async_copy(
                table_sh.at[idx_v.at[0]], rows_v.at[0], g_sem
            ).wait()
            buf = lax.rem(i, 4)
            pltpu.async_copy(
                rows_v.at[buf], out_hbm.at[pl.ds(base + i * CHUNK, CHUNK)], w_sem
            )

            @pl.when(i + 2 < NCHUNKS)
            def _():
                nxt = lax.rem(i + 2, 4)

                @pl.when(i >= 2)
                def _():
                    pltpu.make_async_copy(
                        rows_v.at[0], out_hbm.at[pl.ds(base, CHUNK)], w_sem
                    ).wait()

                pltpu.async_copy(
                    table_sh.at[idx_v.at[i + 2]], rows_v.at[nxt], g_sem
                )

            return carry

        lax.fori_loop(0, NCHUNKS, chunk_step, 0)
        for b in range(4):
            pltpu.make_async_copy(
                rows_v.at[b], out_hbm.at[pl.ds(base, CHUNK)], w_sem
            ).wait()

    return gather_kernel


_kernel_fn = _make_kernel()


@jax.jit
def kernel(span_indices, table):
    idx = span_indices.reshape(NW, NCHUNKS, CHUNK).astype(jnp.int32)
    out = _kernel_fn(idx, table)
    return out.reshape(BATCH, SEQ_LEN, MODEL_DIM)
